# baseline (speedup 1.0000x reference)
import functools

import jax
import jax.numpy as jnp
import numpy as np
from jax import lax
from jax.experimental import pallas as pl
from jax.experimental.pallas import tpu as pltpu
from jax.experimental.pallas import tpu_sc as plsc

_NUM_CLASSES = 1000000
_HIDDEN = 128
_DROPOUT_PROB = 0.1
_SEED = 0
_BATCH = 16384

_INFO = plsc.get_sparse_core_info()
_NC, _NS = _INFO.num_cores, _INFO.num_subcores
_NW = _NC * _NS
_B_PER_W = _BATCH // _NW
_CHUNK = 128
_NCHUNK = _B_PER_W // _CHUNK
_L = 16

def _threefry2x32(k1, k2, x0, x1):
    def rotl(x, d):
        return ((x << np.uint32(d)) | (x >> np.uint32(32 - d))).astype(np.uint32)

    rotations = [(13, 15, 26, 6), (17, 29, 16, 24)]
    ks = [k1, k2, np.uint32(k1 ^ k2 ^ np.uint32(0x1BD11BDA))]
    x0 = (x0 + ks[0]).astype(np.uint32)
    x1 = (x1 + ks[1]).astype(np.uint32)
    for i in range(5):
        for r in rotations[i % 2]:
            x0 = (x0 + x1).astype(np.uint32)
            x1 = rotl(x1, r)
            x1 = (x0 ^ x1).astype(np.uint32)
        x0 = (x0 + ks[(i + 1) % 3]).astype(np.uint32)
        x1 = (x1 + ks[(i + 2) % 3] + np.uint32(i + 1)).astype(np.uint32)
    return x0, x1


def _drop_mask(seed, n, p):
    k1 = np.uint32((seed >> 32) & 0xFFFFFFFF)
    k2 = np.uint32(seed & 0xFFFFFFFF)
    iota = np.arange(n, dtype=np.uint64)
    hi = (iota >> np.uint64(32)).astype(np.uint32)
    lo = (iota & np.uint64(0xFFFFFFFF)).astype(np.uint32)
    b1, b2 = _threefry2x32(k1, k2, hi, lo)
    bits = (b1 ^ b2).astype(np.uint32)
    u = ((bits >> np.uint32(9)) | np.uint32(0x3F800000)).view(np.float32) - np.float32(1.0)
    return u < p


_DROP_MASK = _drop_mask(_SEED, _BATCH, _DROPOUT_PROB).astype(np.int32)

_mesh = plsc.VectorSubcoreMesh(core_axis_name="c", subcore_axis_name="s")


@functools.partial(
    pl.kernel,
    mesh=_mesh,
    out_type=jax.ShapeDtypeStruct((_BATCH, _HIDDEN), jnp.float32),
    scratch_types=[
        pltpu.VMEM((_B_PER_W,), jnp.int32),
        pltpu.VMEM((_B_PER_W,), jnp.int32),
        pltpu.VMEM((_L,), jnp.int32),
        pltpu.VMEM((_B_PER_W, _HIDDEN), jnp.float32),
        pltpu.SemaphoreType.DMA((_NCHUNK,)),
        pltpu.SemaphoreType.DMA((3,)),
        pltpu.SemaphoreType.DMA,
    ],
)
def _gather_kernel(labels_hbm, mask_hbm, train_hbm, table_hbm, out_hbm,
                   idx_v, msk_v, trn_v, rows_v, gsem, ssem, osem):
    wid = lax.axis_index("s") * _NC + lax.axis_index("c")
    base = wid * _B_PER_W
    stages = [
        pltpu.async_copy(labels_hbm.at[pl.ds(base, _B_PER_W)], idx_v, ssem.at[0]),
        pltpu.async_copy(mask_hbm.at[pl.ds(base, _B_PER_W)], msk_v, ssem.at[1]),
        pltpu.async_copy(train_hbm, trn_v, ssem.at[2]),
    ]
    for c in stages:
        c.wait()
    train_on = trn_v[...] != 0
    null_class = jnp.full((_L,), _NUM_CLASSES, jnp.int32)
    for g in range(_B_PER_W // _L):
        sl = pl.ds(g * _L, _L)
        lbl = idx_v[sl]
        dropped = (msk_v[sl] != 0) & train_on
        idx_v[sl] = jnp.where(dropped, null_class, lbl)
    gathers = []
    for j in range(_NCHUNK):
        gathers.append(
            pltpu.async_copy(
                table_hbm.at[idx_v.at[pl.ds(j * _CHUNK, _CHUNK)]],
                rows_v.at[pl.ds(j * _CHUNK, _CHUNK)],
                gsem.at[j],
            )
        )
    outs = []
    for j in range(_NCHUNK):
        gathers[j].wait()
        outs.append(
            pltpu.async_copy(
                rows_v.at[pl.ds(j * _CHUNK, _CHUNK)],
                out_hbm.at[pl.ds(base + j * _CHUNK, _CHUNK)],
                osem,
            )
        )
    for c in outs:
        c.wait()


def kernel(labels, train, table):
    labels = labels.astype(jnp.int32)
    mask = jnp.asarray(_DROP_MASK)
    train16 = jnp.full((_L,), train, dtype=jnp.int32)
    return _gather_kernel(labels, mask, train16, table)


# speedup vs baseline: 1.0529x; 1.0529x over previous
import functools

import jax
import jax.numpy as jnp
import numpy as np
from jax import lax
from jax.experimental import pallas as pl
from jax.experimental.pallas import tpu as pltpu
from jax.experimental.pallas import tpu_sc as plsc

_NUM_CLASSES = 1000000
_HIDDEN = 128
_DROPOUT_PROB = 0.1
_SEED = 0
_BATCH = 16384

_INFO = plsc.get_sparse_core_info()
_NC, _NS = _INFO.num_cores, _INFO.num_subcores
_NW = _NC * _NS
_B_PER_W = _BATCH // _NW
_CHUNK = 128
_NCHUNK = _B_PER_W // _CHUNK
_L = 16

def _threefry2x32(k1, k2, x0, x1):
    def rotl(x, d):
        return ((x << np.uint32(d)) | (x >> np.uint32(32 - d))).astype(np.uint32)

    rotations = [(13, 15, 26, 6), (17, 29, 16, 24)]
    ks = [k1, k2, np.uint32(k1 ^ k2 ^ np.uint32(0x1BD11BDA))]
    x0 = (x0 + ks[0]).astype(np.uint32)
    x1 = (x1 + ks[1]).astype(np.uint32)
    for i in range(5):
        for r in rotations[i % 2]:
            x0 = (x0 + x1).astype(np.uint32)
            x1 = rotl(x1, r)
            x1 = (x0 ^ x1).astype(np.uint32)
        x0 = (x0 + ks[(i + 1) % 3]).astype(np.uint32)
        x1 = (x1 + ks[(i + 2) % 3] + np.uint32(i + 1)).astype(np.uint32)
    return x0, x1


def _drop_mask(seed, n, p):
    k1 = np.uint32((seed >> 32) & 0xFFFFFFFF)
    k2 = np.uint32(seed & 0xFFFFFFFF)
    iota = np.arange(n, dtype=np.uint64)
    hi = (iota >> np.uint64(32)).astype(np.uint32)
    lo = (iota & np.uint64(0xFFFFFFFF)).astype(np.uint32)
    b1, b2 = _threefry2x32(k1, k2, hi, lo)
    bits = (b1 ^ b2).astype(np.uint32)
    u = ((bits >> np.uint32(9)) | np.uint32(0x3F800000)).view(np.float32) - np.float32(1.0)
    return u < p


_DROP_MASK = _drop_mask(_SEED, _BATCH, _DROPOUT_PROB).astype(np.int32)

_mesh = plsc.VectorSubcoreMesh(core_axis_name="c", subcore_axis_name="s")


@functools.partial(
    pl.kernel,
    mesh=_mesh,
    out_type=jax.ShapeDtypeStruct((_BATCH, _HIDDEN), jnp.float32),
    scratch_types=[
        pltpu.VMEM((_B_PER_W,), jnp.int32),
        pltpu.VMEM((_B_PER_W,), jnp.int32),
        pltpu.VMEM((_L,), jnp.int32),
        pltpu.VMEM((_B_PER_W, _HIDDEN), jnp.float32),
        pltpu.SemaphoreType.DMA((_NCHUNK,)),
        pltpu.SemaphoreType.DMA((3,)),
        pltpu.SemaphoreType.DMA,
    ],
)
def _gather_kernel(labels_hbm, mask_hbm, train_hbm, table_hbm, out_hbm,
                   idx_v, msk_v, trn_v, rows_v, gsem, ssem, osem):
    wid = lax.axis_index("s") * _NC + lax.axis_index("c")
    base = wid * _B_PER_W
    lab_c = pltpu.async_copy(labels_hbm.at[pl.ds(base, _B_PER_W)], idx_v, ssem.at[0])
    trn_c = pltpu.async_copy(train_hbm, trn_v.at[pl.ds(0, 1)], ssem.at[2])
    trn_c.wait()
    lab_c.wait()

    train_vec = trn_v[...]

    @pl.when(train_vec[0] != 0)
    def _apply_dropout():
        pltpu.async_copy(mask_hbm.at[pl.ds(base, _B_PER_W)], msk_v, ssem.at[1]).wait()
        null_class = jnp.full((_L,), _NUM_CLASSES, jnp.int32)
        for g in range(_B_PER_W // _L):
            sl = pl.ds(g * _L, _L)
            lbl = idx_v[sl]
            dropped = msk_v[sl] != 0
            idx_v[sl] = jnp.where(dropped, null_class, lbl)
    gathers = []
    for j in range(_NCHUNK):
        gathers.append(
            pltpu.async_copy(
                table_hbm.at[idx_v.at[pl.ds(j * _CHUNK, _CHUNK)]],
                rows_v.at[pl.ds(j * _CHUNK, _CHUNK)],
                gsem.at[j],
            )
        )
    outs = []
    for j in range(_NCHUNK):
        gathers[j].wait()
        outs.append(
            pltpu.async_copy(
                rows_v.at[pl.ds(j * _CHUNK, _CHUNK)],
                out_hbm.at[pl.ds(base + j * _CHUNK, _CHUNK)],
                osem,
            )
        )
    for c in outs:
        c.wait()


def kernel(labels, train, table):
    labels = labels.astype(jnp.int32)
    mask = jnp.asarray(_DROP_MASK)
    train1 = jnp.asarray(train, jnp.int32).reshape(1)
    return _gather_kernel(labels, mask, train1, table)
